# Initial kernel scaffold; baseline (speedup 1.0000x reference)
import jax
import jax.numpy as jnp
from jax.experimental import pallas as pl


def kernel(feat, edge_index, e_feat, W, edge_emb_weight):
    raise NotImplementedError("write your pallas kernel here")


# speedup vs baseline: 3.1370x; 3.1370x over previous
import functools

import jax
import jax.numpy as jnp
from jax import lax
from jax.experimental import pallas as pl
from jax.experimental.pallas import tpu as pltpu
from jax.experimental.pallas import tpu_sc as plsc

N = 10000
E = 320000
D_IN = 128
H = 8
D_OUT = 16
HD = H * D_OUT
NT = 16
DROP_BLOCKS = 4

NC = 2
NS = 16
L = 16

HALF = HD // NC
HPC = H // NC
ACCW = 2 * HALF
EPT = E // NS
K = 40
NCH = EPT // K
EPA = E // NC // NS
NCA = EPA // K
BS = 80
NBLK = N // BS
AW = H * DROP_BLOCKS


def _mm_body(f_ref, w_ref, o_ref):
    o_ref[...] = jnp.dot(f_ref[...], w_ref[...], preferred_element_type=jnp.float32)


def _project(feat, Wt):
    return pl.pallas_call(
        _mm_body,
        grid=(NBLK,),
        in_specs=[
            pl.BlockSpec((BS, D_IN), lambda nb: (nb, 0)),
            pl.BlockSpec((D_IN, HD), lambda nb: (0, 0)),
        ],
        out_specs=pl.BlockSpec((BS, HD), lambda nb: (nb, 0)),
        out_shape=jax.ShapeDtypeStruct((N, HD), jnp.float32),
    )(feat, Wt)


TB = 2560


def _tr_body(i_ref, o_ref):
    o_ref[...] = i_ref[...].T


def _transpose_attn(attnE):
    return pl.pallas_call(
        _tr_body,
        grid=(E // TB,),
        in_specs=[pl.BlockSpec((TB, AW), lambda b: (b, 0))],
        out_specs=pl.BlockSpec((AW, TB), lambda b: (0, b)),
        out_shape=jax.ShapeDtypeStruct((AW, E), jnp.float32),
    )(attnE)


def _sc_body(fsall, esrc, edst, ef, embT, rst2, attnf,
             acc, wtabS, embv, eflat, wrow,
             srcb, dstb, typb, gbuf, wbuf, sbuf, abuf, nbuf, rbuf, sem):
    cid = lax.axis_index("c")
    sid = lax.axis_index("s")
    lane = lax.iota(jnp.int32, L)
    zv = jnp.zeros((L,), jnp.float32)

    pltpu.sync_copy(embT, embv)
    for h in range(H):
        v = embv[h]
        m = v
        for sh in (1, 2, 4, 8):
            perm = jnp.bitwise_xor(lane, jnp.int32(sh))
            m = jnp.maximum(m, m.at[perm].get(mode="promise_in_bounds"))
        e = jnp.exp(v - m)
        eflat[pl.ds(h * L, L)] = e

    cf = jnp.broadcast_to(cid.astype(jnp.float32), (L,))
    eh_list = [eflat[pl.ds(h * L, L)] for h in range(H)]
    lane4 = lax.shift_right_logical(lane, 2)

    def build_t(t, _):
        tvec = jnp.broadcast_to(t, (L,))
        base = t * ACCW
        bh = [eh_list[h].at[tvec].get(mode="promise_in_bounds")
              for h in range(H)]
        for j in range(HPC):
            own = bh[j] * (1.0 - cf) + bh[HPC + j] * cf
            wrow[pl.ds(base + j * L, L)] = own
        dv0 = zv
        dv1 = zv
        for j in range(HPC):
            dv0 = jnp.where(lane4 == j, bh[j], dv0)
            dv1 = jnp.where(lane4 == j, bh[HPC + j], dv1)
        wrow[pl.ds(base + HALF, L)] = dv0
        wrow[pl.ds(base + HALF + L, L)] = dv1
        return 0
    lax.fori_loop(0, NT, build_t, 0)

    @pl.when(sid == 0)
    def _():
        def wcopy(t, _):
            pltpu.sync_copy(wrow.at[pl.ds(t * ACCW, ACCW)], wtabS.at[t])
            return 0
        lax.fori_loop(0, NT, wcopy, 0)

    def zrow(i, _):
        for j5 in range(ACCW // L):
            nbuf[i, pl.ds(j5 * L, L)] = zv
        return 0
    lax.fori_loop(0, 25, zrow, 0)

    def zblk(b, _):
        pltpu.sync_copy(nbuf.at[pl.ds(0, 25)], acc.at[pl.ds(sid * 625 + b * 25, 25)])
        return 0
    lax.fori_loop(0, 25, zblk, 0)
    plsc.subcore_barrier()

    cbase = cid * HALF

    def echunk(i, _):
        off = sid * EPT + i * K
        pltpu.sync_copy(esrc.at[pl.ds(off, K)], srcb)
        pltpu.sync_copy(edst.at[pl.ds(off, K)], dstb)
        pltpu.sync_copy(ef.at[pl.ds(off, K)], typb)
        pltpu.async_copy(wtabS.at[typb], wbuf, sem).wait()
        pltpu.async_copy(fsall.at[srcb], gbuf, sem).wait()

        def emul(p, _):
            for j in range(HPC):
                sbuf[p, pl.ds(j * L, L)] = (
                    gbuf[p, pl.ds(cbase + j * L, L)] * wbuf[p, pl.ds(j * L, L)])
            sbuf[p, pl.ds(HALF, L)] = wbuf[p, pl.ds(HALF, L)]
            sbuf[p, pl.ds(HALF + L, L)] = wbuf[p, pl.ds(HALF + L, L)]
            return 0
        lax.fori_loop(0, K, emul, 0)
        pltpu.sync_copy(sbuf, acc.at[dstb], add=True)
        return 0
    lax.fori_loop(0, NCH, echunk, 0)
    plsc.subcore_barrier()

    doff = HALF + cid * L

    def rblk(b, _):
        @pl.when((b % NS) == sid)
        def _():
            pltpu.sync_copy(acc.at[pl.ds(b * BS, BS)], nbuf)

            def rrow(p, _):
                dv = nbuf[p, pl.ds(doff, L)]
                for j in range(HPC):
                    db = dv.at[jnp.broadcast_to(jnp.int32(4 * j), (L,))].get(
                        mode="promise_in_bounds")
                    num = nbuf[p, pl.ds(j * L, L)]
                    rbuf[pl.ds(p * HALF + j * L, L)] = num / jnp.maximum(db, 1e-30)
                return 0
            lax.fori_loop(0, BS, rrow, 0)
            pltpu.sync_copy(
                rbuf, rst2.at[pl.ds((cid * N + b * BS) * HALF, BS * HALF)])
        return 0
    lax.fori_loop(0, NBLK, rblk, 0)
    plsc.subcore_barrier()

    def achunk(i, _):
        off = cid * (E // NC) + sid * EPA + i * K
        pltpu.sync_copy(edst.at[pl.ds(off, K)], dstb)
        pltpu.sync_copy(ef.at[pl.ds(off, K)], typb)
        pltpu.async_copy(wtabS.at[typb], wbuf, sem).wait()
        pltpu.async_copy(acc.at[dstb], sbuf, sem).wait()

        def arow(p, _):
            abuf[pl.ds(p * AW, L)] = (
                wbuf[p, pl.ds(HALF, L)] / sbuf[p, pl.ds(HALF, L)])
            abuf[pl.ds(p * AW + L, L)] = (
                wbuf[p, pl.ds(HALF + L, L)] / sbuf[p, pl.ds(HALF + L, L)])
            return 0
        lax.fori_loop(0, K, arow, 0)
        pltpu.sync_copy(abuf, attnf.at[pl.ds(off * AW, K * AW)])
        return 0
    lax.fori_loop(0, NCA, achunk, 0)


_sc_call = functools.partial(
    pl.kernel,
    out_type=[
        jax.ShapeDtypeStruct((NC * N * HALF,), jnp.float32),
        jax.ShapeDtypeStruct((E * AW,), jnp.float32),
    ],
    mesh=plsc.VectorSubcoreMesh(core_axis_name="c", subcore_axis_name="s"),
    scratch_types=[
        pltpu.VMEM_SHARED((N, ACCW), jnp.float32),
        pltpu.VMEM_SHARED((NT, ACCW), jnp.float32),
        pltpu.VMEM((H, NT), jnp.float32),
        pltpu.VMEM((H * L,), jnp.float32),
        pltpu.VMEM((NT * ACCW,), jnp.float32),
        pltpu.VMEM((K,), jnp.int32),
        pltpu.VMEM((K,), jnp.int32),
        pltpu.VMEM((K,), jnp.int32),
        pltpu.VMEM((K, HD), jnp.float32),
        pltpu.VMEM((K, ACCW), jnp.float32),
        pltpu.VMEM((K, ACCW), jnp.float32),
        pltpu.VMEM((K * AW,), jnp.float32),
        pltpu.VMEM((BS, ACCW), jnp.float32),
        pltpu.VMEM((BS * HALF,), jnp.float32),
        pltpu.SemaphoreType.DMA,
    ],
)(_sc_body)


def kernel(feat, edge_index, e_feat, W, edge_emb_weight):
    fsall = _project(feat, W.T)
    embT = edge_emb_weight.T.astype(jnp.float32)
    rst2f, attnf = _sc_call(fsall, edge_index[0], edge_index[1], e_feat, embT)
    rr = rst2f.reshape(NC, N, HALF)
    rst = jnp.concatenate([rr[0], rr[1]], axis=1).reshape(N, H, D_OUT)
    attn32 = _transpose_attn(attnf.reshape(E, AW))
    attn = jnp.transpose(
        attn32.reshape(H, DROP_BLOCKS, 1, E), (3, 0, 1, 2))
    return (rst, attn)


# speedup vs baseline: 3.3395x; 1.0646x over previous
import functools

import jax
import jax.numpy as jnp
from jax import lax
from jax.experimental import pallas as pl
from jax.experimental.pallas import tpu as pltpu
from jax.experimental.pallas import tpu_sc as plsc

N = 10000
E = 320000
D_IN = 128
H = 8
D_OUT = 16
HD = H * D_OUT
NT = 16
DROP_BLOCKS = 4

NC = 2
NS = 16
L = 16

HALF = HD // NC
HPC = H // NC
ACCW = 2 * HALF
EPT = E // NS
K = 40
NCH = EPT // K
EPA = E // NC // NS
NCA = EPA // K
BS = 80
NBLK = N // BS
AW = H * DROP_BLOCKS


def _mm_body(f_ref, w_ref, o_ref):
    o_ref[...] = jnp.dot(f_ref[...], w_ref[...], preferred_element_type=jnp.float32)


def _project(feat, Wt):
    return pl.pallas_call(
        _mm_body,
        grid=(NBLK,),
        in_specs=[
            pl.BlockSpec((BS, D_IN), lambda nb: (nb, 0)),
            pl.BlockSpec((D_IN, HD), lambda nb: (0, 0)),
        ],
        out_specs=pl.BlockSpec((BS, HD), lambda nb: (nb, 0)),
        out_shape=jax.ShapeDtypeStruct((N, HD), jnp.float32),
    )(feat, Wt)


TB = 2560


def _tr_body(i_ref, o_ref):
    o_ref[...] = i_ref[...].T


def _transpose_attn(attnE):
    return pl.pallas_call(
        _tr_body,
        grid=(E // TB,),
        in_specs=[pl.BlockSpec((TB, AW), lambda b: (b, 0))],
        out_specs=pl.BlockSpec((AW, TB), lambda b: (0, b)),
        out_shape=jax.ShapeDtypeStruct((AW, E), jnp.float32),
    )(attnE)


def _sc_body(fsall, esrc, edst, ef, embT, rst2, attnf,
             acc, wtabS, embv, eflat, wrow,
             srcb, dstb, typb, gbuf, wbuf, sbuf, abuf, nbuf, rbuf, sem, sem2):
    cid = lax.axis_index("c")
    sid = lax.axis_index("s")
    lane = lax.iota(jnp.int32, L)
    zv = jnp.zeros((L,), jnp.float32)

    pltpu.sync_copy(embT, embv)
    for h in range(H):
        v = embv[h]
        m = v
        for sh in (1, 2, 4, 8):
            perm = jnp.bitwise_xor(lane, jnp.int32(sh))
            m = jnp.maximum(m, m.at[perm].get(mode="promise_in_bounds"))
        e = jnp.exp(v - m)
        eflat[pl.ds(h * L, L)] = e

    cf = jnp.broadcast_to(cid.astype(jnp.float32), (L,))
    eh_list = [eflat[pl.ds(h * L, L)] for h in range(H)]
    lane4 = lax.shift_right_logical(lane, 2)

    def build_t(t, _):
        tvec = jnp.broadcast_to(t, (L,))
        base = t * ACCW
        bh = [eh_list[h].at[tvec].get(mode="promise_in_bounds")
              for h in range(H)]
        for j in range(HPC):
            own = bh[j] * (1.0 - cf) + bh[HPC + j] * cf
            wrow[pl.ds(base + j * L, L)] = own
        dv0 = zv
        dv1 = zv
        for j in range(HPC):
            dv0 = jnp.where(lane4 == j, bh[j], dv0)
            dv1 = jnp.where(lane4 == j, bh[HPC + j], dv1)
        wrow[pl.ds(base + HALF, L)] = dv0
        wrow[pl.ds(base + HALF + L, L)] = dv1
        return 0
    lax.fori_loop(0, NT, build_t, 0)

    @pl.when(sid == 0)
    def _():
        def wcopy(t, _):
            pltpu.sync_copy(wrow.at[pl.ds(t * ACCW, ACCW)], wtabS.at[t])
            return 0
        lax.fori_loop(0, NT, wcopy, 0)

    def zrow(i, _):
        for j5 in range(ACCW // L):
            nbuf[i, pl.ds(j5 * L, L)] = zv
        return 0
    lax.fori_loop(0, 25, zrow, 0)

    def zblk(b, _):
        pltpu.sync_copy(nbuf.at[pl.ds(0, 25)], acc.at[pl.ds(sid * 625 + b * 25, 25)])
        return 0
    lax.fori_loop(0, 25, zblk, 0)
    plsc.subcore_barrier()

    cbase = cid * HALF

    def echunk(i, _):
        off = sid * EPT + i * K
        pltpu.sync_copy(esrc.at[pl.ds(off, K)], srcb)
        pltpu.sync_copy(edst.at[pl.ds(off, K)], dstb)
        pltpu.sync_copy(ef.at[pl.ds(off, K)], typb)
        wcp = pltpu.async_copy(wtabS.at[typb], wbuf, sem2)
        pltpu.async_copy(fsall.at[srcb], gbuf, sem).wait()
        wcp.wait()

        def emul(p, _):
            for j in range(HPC):
                sbuf[p, pl.ds(j * L, L)] = (
                    gbuf[p, pl.ds(cbase + j * L, L)] * wbuf[p, pl.ds(j * L, L)])
            sbuf[p, pl.ds(HALF, L)] = wbuf[p, pl.ds(HALF, L)]
            sbuf[p, pl.ds(HALF + L, L)] = wbuf[p, pl.ds(HALF + L, L)]
            return 0
        lax.fori_loop(0, K, emul, 0)
        pltpu.sync_copy(sbuf, acc.at[dstb], add=True)
        return 0
    lax.fori_loop(0, NCH, echunk, 0)
    plsc.subcore_barrier()

    doff = HALF + cid * L

    def rblk(b, _):
        @pl.when((b % NS) == sid)
        def _():
            pltpu.sync_copy(acc.at[pl.ds(b * BS, BS)], nbuf)

            def rrow(p, _):
                dv = nbuf[p, pl.ds(doff, L)]
                for j in range(HPC):
                    db = dv.at[jnp.broadcast_to(jnp.int32(4 * j), (L,))].get(
                        mode="promise_in_bounds")
                    num = nbuf[p, pl.ds(j * L, L)]
                    rbuf[pl.ds(p * HALF + j * L, L)] = num / jnp.maximum(db, 1e-30)
                return 0
            lax.fori_loop(0, BS, rrow, 0)
            pltpu.sync_copy(
                rbuf, rst2.at[pl.ds((cid * N + b * BS) * HALF, BS * HALF)])
        return 0
    lax.fori_loop(0, NBLK, rblk, 0)
    plsc.subcore_barrier()

    def achunk(i, _):
        off = cid * (E // NC) + sid * EPA + i * K
        pltpu.sync_copy(edst.at[pl.ds(off, K)], dstb)
        pltpu.sync_copy(ef.at[pl.ds(off, K)], typb)
        wcp = pltpu.async_copy(wtabS.at[typb], wbuf, sem2)
        pltpu.async_copy(acc.at[dstb], sbuf, sem).wait()
        wcp.wait()

        def arow(p, _):
            abuf[pl.ds(p * AW, L)] = (
                wbuf[p, pl.ds(HALF, L)] / sbuf[p, pl.ds(HALF, L)])
            abuf[pl.ds(p * AW + L, L)] = (
                wbuf[p, pl.ds(HALF + L, L)] / sbuf[p, pl.ds(HALF + L, L)])
            return 0
        lax.fori_loop(0, K, arow, 0)
        pltpu.sync_copy(abuf, attnf.at[pl.ds(off * AW, K * AW)])
        return 0
    lax.fori_loop(0, NCA, achunk, 0)


_sc_call = functools.partial(
    pl.kernel,
    out_type=[
        jax.ShapeDtypeStruct((NC * N * HALF,), jnp.float32),
        jax.ShapeDtypeStruct((E * AW,), jnp.float32),
    ],
    mesh=plsc.VectorSubcoreMesh(core_axis_name="c", subcore_axis_name="s"),
    scratch_types=[
        pltpu.VMEM_SHARED((N, ACCW), jnp.float32),
        pltpu.VMEM_SHARED((NT, ACCW), jnp.float32),
        pltpu.VMEM((H, NT), jnp.float32),
        pltpu.VMEM((H * L,), jnp.float32),
        pltpu.VMEM((NT * ACCW,), jnp.float32),
        pltpu.VMEM((K,), jnp.int32),
        pltpu.VMEM((K,), jnp.int32),
        pltpu.VMEM((K,), jnp.int32),
        pltpu.VMEM((K, HD), jnp.float32),
        pltpu.VMEM((K, ACCW), jnp.float32),
        pltpu.VMEM((K, ACCW), jnp.float32),
        pltpu.VMEM((K * AW,), jnp.float32),
        pltpu.VMEM((BS, ACCW), jnp.float32),
        pltpu.VMEM((BS * HALF,), jnp.float32),
        pltpu.SemaphoreType.DMA,
        pltpu.SemaphoreType.DMA,
    ],
)(_sc_body)


def kernel(feat, edge_index, e_feat, W, edge_emb_weight):
    fsall = _project(feat, W.T)
    embT = edge_emb_weight.T.astype(jnp.float32)
    rst2f, attnf = _sc_call(fsall, edge_index[0], edge_index[1], e_feat, embT)
    rr = rst2f.reshape(NC, N, HALF)
    rst = jnp.concatenate([rr[0], rr[1]], axis=1).reshape(N, H, D_OUT)
    attn32 = _transpose_attn(attnf.reshape(E, AW))
    attn = jnp.transpose(
        attn32.reshape(H, DROP_BLOCKS, 1, E), (3, 0, 1, 2))
    return (rst, attn)


# speedup vs baseline: 3.4217x; 1.0246x over previous
import functools

import jax
import jax.numpy as jnp
from jax import lax
from jax.experimental import pallas as pl
from jax.experimental.pallas import tpu as pltpu
from jax.experimental.pallas import tpu_sc as plsc

N = 10000
E = 320000
D_IN = 128
H = 8
D_OUT = 16
HD = H * D_OUT
NT = 16
DROP_BLOCKS = 4

NC = 2
NS = 16
L = 16

HALF = HD // NC
HPC = H // NC
ACCW = 2 * HALF
EPT = E // NS
K = 40
NCH = EPT // K
EPA = E // NC // NS
NCA = EPA // K
BS = 80
NBLK = N // BS
AW = H * DROP_BLOCKS


def _mm_body(f_ref, w_ref, o_ref):
    o_ref[...] = jnp.dot(f_ref[...], w_ref[...], preferred_element_type=jnp.float32)


def _project(feat, Wt):
    return pl.pallas_call(
        _mm_body,
        grid=(NBLK,),
        in_specs=[
            pl.BlockSpec((BS, D_IN), lambda nb: (nb, 0)),
            pl.BlockSpec((D_IN, HD), lambda nb: (0, 0)),
        ],
        out_specs=pl.BlockSpec((BS, HD), lambda nb: (nb, 0)),
        out_shape=jax.ShapeDtypeStruct((N, HD), jnp.float32),
    )(feat, Wt)


TB = 2560


def _tr_body(i_ref, o_ref):
    o_ref[...] = i_ref[...].T


def _transpose_attn(attnE):
    return pl.pallas_call(
        _tr_body,
        grid=(E // TB,),
        in_specs=[pl.BlockSpec((TB, AW), lambda b: (b, 0))],
        out_specs=pl.BlockSpec((AW, TB), lambda b: (0, b)),
        out_shape=jax.ShapeDtypeStruct((AW, E), jnp.float32),
    )(attnE)


def _sc_body(fsall, esrc, edst, ef, embT, rst2, attnf,
             acc, wtabS, embv, eflat, wrow,
             srcb, dstb, typb, gbuf, wbuf, sbuf, dstb2, sbuf2, abuf, nbuf,
             rbuf, sem, sem2, sem3, sem4):
    cid = lax.axis_index("c")
    sid = lax.axis_index("s")
    lane = lax.iota(jnp.int32, L)
    zv = jnp.zeros((L,), jnp.float32)

    pltpu.sync_copy(embT, embv)
    for h in range(H):
        v = embv[h]
        m = v
        for sh in (1, 2, 4, 8):
            perm = jnp.bitwise_xor(lane, jnp.int32(sh))
            m = jnp.maximum(m, m.at[perm].get(mode="promise_in_bounds"))
        e = jnp.exp(v - m)
        eflat[pl.ds(h * L, L)] = e

    cf = jnp.broadcast_to(cid.astype(jnp.float32), (L,))
    eh_list = [eflat[pl.ds(h * L, L)] for h in range(H)]
    lane4 = lax.shift_right_logical(lane, 2)

    def build_t(t, _):
        tvec = jnp.broadcast_to(t, (L,))
        base = t * ACCW
        bh = [eh_list[h].at[tvec].get(mode="promise_in_bounds")
              for h in range(H)]
        for j in range(HPC):
            own = bh[j] * (1.0 - cf) + bh[HPC + j] * cf
            wrow[pl.ds(base + j * L, L)] = own
        dv0 = zv
        dv1 = zv
        for j in range(HPC):
            dv0 = jnp.where(lane4 == j, bh[j], dv0)
            dv1 = jnp.where(lane4 == j, bh[HPC + j], dv1)
        wrow[pl.ds(base + HALF, L)] = dv0
        wrow[pl.ds(base + HALF + L, L)] = dv1
        return 0
    lax.fori_loop(0, NT, build_t, 0)

    @pl.when(sid == 0)
    def _():
        def wcopy(t, _):
            pltpu.sync_copy(wrow.at[pl.ds(t * ACCW, ACCW)], wtabS.at[t])
            return 0
        lax.fori_loop(0, NT, wcopy, 0)

    def zrow(i, _):
        for j5 in range(ACCW // L):
            nbuf[i, pl.ds(j5 * L, L)] = zv
        return 0
    lax.fori_loop(0, 25, zrow, 0)

    def zblk(b, _):
        pltpu.sync_copy(nbuf.at[pl.ds(0, 25)], acc.at[pl.ds(sid * 625 + b * 25, 25)])
        return 0
    lax.fori_loop(0, 25, zblk, 0)
    plsc.subcore_barrier()

    cbase = cid * HALF

    def emul_into(sb, p, _):
        for j in range(HPC):
            sb[p, pl.ds(j * L, L)] = (
                gbuf[p, pl.ds(cbase + j * L, L)] * wbuf[p, pl.ds(j * L, L)])
        sb[p, pl.ds(HALF, L)] = wbuf[p, pl.ds(HALF, L)]
        sb[p, pl.ds(HALF + L, L)] = wbuf[p, pl.ds(HALF + L, L)]
        return 0

    def echunk(i, _):
        @pl.when(i > 0)
        def _():
            pltpu.make_async_copy(sbuf, acc.at[dstb], sem3).wait()
            pltpu.make_async_copy(sbuf2, acc.at[dstb2], sem4).wait()

        offA = sid * EPT + (2 * i) * K
        pltpu.sync_copy(esrc.at[pl.ds(offA, K)], srcb)
        pltpu.sync_copy(edst.at[pl.ds(offA, K)], dstb)
        pltpu.sync_copy(ef.at[pl.ds(offA, K)], typb)
        wcp = pltpu.async_copy(wtabS.at[typb], wbuf, sem2)
        pltpu.async_copy(fsall.at[srcb], gbuf, sem).wait()
        wcp.wait()
        lax.fori_loop(0, K, functools.partial(emul_into, sbuf), 0)
        pltpu.async_copy(sbuf, acc.at[dstb], sem3, add=True)

        offB = offA + K
        pltpu.sync_copy(esrc.at[pl.ds(offB, K)], srcb)
        pltpu.sync_copy(edst.at[pl.ds(offB, K)], dstb2)
        pltpu.sync_copy(ef.at[pl.ds(offB, K)], typb)
        wcp2 = pltpu.async_copy(wtabS.at[typb], wbuf, sem2)
        pltpu.async_copy(fsall.at[srcb], gbuf, sem).wait()
        wcp2.wait()
        lax.fori_loop(0, K, functools.partial(emul_into, sbuf2), 0)
        pltpu.async_copy(sbuf2, acc.at[dstb2], sem4, add=True)
        return 0
    lax.fori_loop(0, NCH // 2, echunk, 0)
    pltpu.make_async_copy(sbuf, acc.at[dstb], sem3).wait()
    pltpu.make_async_copy(sbuf2, acc.at[dstb2], sem4).wait()
    plsc.subcore_barrier()

    doff = HALF + cid * L

    def rblk(b, _):
        @pl.when((b % NS) == sid)
        def _():
            pltpu.sync_copy(acc.at[pl.ds(b * BS, BS)], nbuf)

            def rrow(p, _):
                dv = nbuf[p, pl.ds(doff, L)]
                for j in range(HPC):
                    db = dv.at[jnp.broadcast_to(jnp.int32(4 * j), (L,))].get(
                        mode="promise_in_bounds")
                    num = nbuf[p, pl.ds(j * L, L)]
                    rbuf[pl.ds(p * HALF + j * L, L)] = num / jnp.maximum(db, 1e-30)
                return 0
            lax.fori_loop(0, BS, rrow, 0)
            pltpu.sync_copy(
                rbuf, rst2.at[pl.ds((cid * N + b * BS) * HALF, BS * HALF)])
        return 0
    lax.fori_loop(0, NBLK, rblk, 0)
    plsc.subcore_barrier()

    def achunk(i, _):
        off = cid * (E // NC) + sid * EPA + i * K
        pltpu.sync_copy(edst.at[pl.ds(off, K)], dstb)
        pltpu.sync_copy(ef.at[pl.ds(off, K)], typb)
        wcp = pltpu.async_copy(wtabS.at[typb], wbuf, sem2)
        pltpu.async_copy(acc.at[dstb], sbuf, sem).wait()
        wcp.wait()

        def arow(p, _):
            abuf[pl.ds(p * AW, L)] = (
                wbuf[p, pl.ds(HALF, L)] / sbuf[p, pl.ds(HALF, L)])
            abuf[pl.ds(p * AW + L, L)] = (
                wbuf[p, pl.ds(HALF + L, L)] / sbuf[p, pl.ds(HALF + L, L)])
            return 0
        lax.fori_loop(0, K, arow, 0)
        pltpu.sync_copy(abuf, attnf.at[pl.ds(off * AW, K * AW)])
        return 0
    lax.fori_loop(0, NCA, achunk, 0)


_sc_call = functools.partial(
    pl.kernel,
    out_type=[
        jax.ShapeDtypeStruct((NC * N * HALF,), jnp.float32),
        jax.ShapeDtypeStruct((E * AW,), jnp.float32),
    ],
    mesh=plsc.VectorSubcoreMesh(core_axis_name="c", subcore_axis_name="s"),
    scratch_types=[
        pltpu.VMEM_SHARED((N, ACCW), jnp.float32),
        pltpu.VMEM_SHARED((NT, ACCW), jnp.float32),
        pltpu.VMEM((H, NT), jnp.float32),
        pltpu.VMEM((H * L,), jnp.float32),
        pltpu.VMEM((NT * ACCW,), jnp.float32),
        pltpu.VMEM((K,), jnp.int32),
        pltpu.VMEM((K,), jnp.int32),
        pltpu.VMEM((K,), jnp.int32),
        pltpu.VMEM((K, HD), jnp.float32),
        pltpu.VMEM((K, ACCW), jnp.float32),
        pltpu.VMEM((K, ACCW), jnp.float32),
        pltpu.VMEM((K,), jnp.int32),
        pltpu.VMEM((K, ACCW), jnp.float32),
        pltpu.VMEM((K * AW,), jnp.float32),
        pltpu.VMEM((BS, ACCW), jnp.float32),
        pltpu.VMEM((BS * HALF,), jnp.float32),
        pltpu.SemaphoreType.DMA,
        pltpu.SemaphoreType.DMA,
        pltpu.SemaphoreType.DMA,
        pltpu.SemaphoreType.DMA,
    ],
)(_sc_body)


def kernel(feat, edge_index, e_feat, W, edge_emb_weight):
    fsall = _project(feat, W.T)
    embT = edge_emb_weight.T.astype(jnp.float32)
    rst2f, attnf = _sc_call(fsall, edge_index[0], edge_index[1], e_feat, embT)
    rr = rst2f.reshape(NC, N, HALF)
    rst = jnp.concatenate([rr[0], rr[1]], axis=1).reshape(N, H, D_OUT)
    attn32 = _transpose_attn(attnf.reshape(E, AW))
    attn = jnp.transpose(
        attn32.reshape(H, DROP_BLOCKS, 1, E), (3, 0, 1, 2))
    return (rst, attn)


# speedup vs baseline: 3.4493x; 1.0081x over previous
import functools

import jax
import jax.numpy as jnp
from jax import lax
from jax.experimental import pallas as pl
from jax.experimental.pallas import tpu as pltpu
from jax.experimental.pallas import tpu_sc as plsc

N = 10000
E = 320000
D_IN = 128
H = 8
D_OUT = 16
HD = H * D_OUT
NT = 16
DROP_BLOCKS = 4

NC = 2
NS = 16
L = 16

HALF = HD // NC
HPC = H // NC
ACCW = 2 * HALF
EPT = E // NS
K = 40
NCH = EPT // K
EPA = E // NC // NS
NCA = EPA // K
BS = 80
NBLK = N // BS
AW = H * DROP_BLOCKS


def _mm_body(f_ref, w_ref, o_ref):
    o_ref[...] = jnp.dot(f_ref[...], w_ref[...], preferred_element_type=jnp.float32)


def _project(feat, Wt):
    return pl.pallas_call(
        _mm_body,
        grid=(NBLK,),
        in_specs=[
            pl.BlockSpec((BS, D_IN), lambda nb: (nb, 0)),
            pl.BlockSpec((D_IN, HD), lambda nb: (0, 0)),
        ],
        out_specs=pl.BlockSpec((BS, HD), lambda nb: (nb, 0)),
        out_shape=jax.ShapeDtypeStruct((N, HD), jnp.float32),
    )(feat, Wt)


TB = 2560


def _tr_body(i_ref, o_ref):
    o_ref[...] = i_ref[...].T


def _transpose_attn(attnE):
    return pl.pallas_call(
        _tr_body,
        grid=(E // TB,),
        in_specs=[pl.BlockSpec((TB, AW), lambda b: (b, 0))],
        out_specs=pl.BlockSpec((AW, TB), lambda b: (0, b)),
        out_shape=jax.ShapeDtypeStruct((AW, E), jnp.float32),
    )(attnE)


def _sc_body(fsall, esrc, edst, ef, embT, rst2, attnf,
             acc, wtabS, embv, eflat, wrow,
             srcb, dstb, typb, gbuf, wbuf, sbuf, dstb2, sbuf2, abuf, nbuf,
             rbuf, sem, sem2, sem3, sem4):
    cid = lax.axis_index("c")
    sid = lax.axis_index("s")
    lane = lax.iota(jnp.int32, L)
    zv = jnp.zeros((L,), jnp.float32)

    pltpu.sync_copy(embT, embv)
    for h in range(H):
        v = embv[h]
        m = v
        for sh in (1, 2, 4, 8):
            perm = jnp.bitwise_xor(lane, jnp.int32(sh))
            m = jnp.maximum(m, m.at[perm].get(mode="promise_in_bounds"))
        e = jnp.exp(v - m)
        eflat[pl.ds(h * L, L)] = e

    cf = jnp.broadcast_to(cid.astype(jnp.float32), (L,))
    eh_list = [eflat[pl.ds(h * L, L)] for h in range(H)]
    lane4 = lax.shift_right_logical(lane, 2)

    def build_t(t, _):
        tvec = jnp.broadcast_to(t, (L,))
        base = t * ACCW
        bh = [eh_list[h].at[tvec].get(mode="promise_in_bounds")
              for h in range(H)]
        for j in range(HPC):
            own = bh[j] * (1.0 - cf) + bh[HPC + j] * cf
            wrow[pl.ds(base + j * L, L)] = own
        dv0 = zv
        dv1 = zv
        for j in range(HPC):
            dv0 = jnp.where(lane4 == j, bh[j], dv0)
            dv1 = jnp.where(lane4 == j, bh[HPC + j], dv1)
        wrow[pl.ds(base + HALF, L)] = dv0
        wrow[pl.ds(base + HALF + L, L)] = dv1
        return 0
    lax.fori_loop(0, NT, build_t, 0)

    @pl.when(sid == 0)
    def _():
        def wcopy(t, _):
            pltpu.sync_copy(wrow.at[pl.ds(t * ACCW, ACCW)], wtabS.at[t])
            return 0
        lax.fori_loop(0, NT, wcopy, 0)

    def zrow(i, _):
        for j5 in range(ACCW // L):
            nbuf[i, pl.ds(j5 * L, L)] = zv
        return 0
    lax.fori_loop(0, 25, zrow, 0)

    def zblk(b, _):
        pltpu.sync_copy(nbuf.at[pl.ds(0, 25)], acc.at[pl.ds(sid * 625 + b * 25, 25)])
        return 0
    lax.fori_loop(0, 25, zblk, 0)
    plsc.subcore_barrier()

    cbase = cid * HALF

    def emul_into(sb, q, _):
        for u in range(4):
            p = q * 4 + u
            for j in range(HPC):
                sb[p, pl.ds(j * L, L)] = (
                    gbuf[p, pl.ds(cbase + j * L, L)] * wbuf[p, pl.ds(j * L, L)])
            sb[p, pl.ds(HALF, L)] = wbuf[p, pl.ds(HALF, L)]
            sb[p, pl.ds(HALF + L, L)] = wbuf[p, pl.ds(HALF + L, L)]
        return 0

    def echunk(i, _):
        @pl.when(i > 0)
        def _():
            pltpu.make_async_copy(sbuf, acc.at[dstb], sem3).wait()
            pltpu.make_async_copy(sbuf2, acc.at[dstb2], sem4).wait()

        offA = sid * EPT + (2 * i) * K
        pltpu.sync_copy(esrc.at[pl.ds(offA, K)], srcb)
        pltpu.sync_copy(edst.at[pl.ds(offA, K)], dstb)
        pltpu.sync_copy(ef.at[pl.ds(offA, K)], typb)
        wcp = pltpu.async_copy(wtabS.at[typb], wbuf, sem2)
        pltpu.async_copy(fsall.at[srcb], gbuf, sem).wait()
        wcp.wait()
        lax.fori_loop(0, K // 4, functools.partial(emul_into, sbuf), 0)
        pltpu.async_copy(sbuf, acc.at[dstb], sem3, add=True)

        offB = offA + K
        pltpu.sync_copy(esrc.at[pl.ds(offB, K)], srcb)
        pltpu.sync_copy(edst.at[pl.ds(offB, K)], dstb2)
        pltpu.sync_copy(ef.at[pl.ds(offB, K)], typb)
        wcp2 = pltpu.async_copy(wtabS.at[typb], wbuf, sem2)
        pltpu.async_copy(fsall.at[srcb], gbuf, sem).wait()
        wcp2.wait()
        lax.fori_loop(0, K // 4, functools.partial(emul_into, sbuf2), 0)
        pltpu.async_copy(sbuf2, acc.at[dstb2], sem4, add=True)
        return 0
    lax.fori_loop(0, NCH // 2, echunk, 0)
    pltpu.make_async_copy(sbuf, acc.at[dstb], sem3).wait()
    pltpu.make_async_copy(sbuf2, acc.at[dstb2], sem4).wait()
    plsc.subcore_barrier()

    doff = HALF + cid * L

    def rblk(b, _):
        @pl.when((b % NS) == sid)
        def _():
            pltpu.sync_copy(acc.at[pl.ds(b * BS, BS)], nbuf)

            def rrow(p, _):
                dv = nbuf[p, pl.ds(doff, L)]
                for j in range(HPC):
                    db = dv.at[jnp.broadcast_to(jnp.int32(4 * j), (L,))].get(
                        mode="promise_in_bounds")
                    num = nbuf[p, pl.ds(j * L, L)]
                    rbuf[pl.ds(p * HALF + j * L, L)] = num / jnp.maximum(db, 1e-30)
                return 0
            lax.fori_loop(0, BS, rrow, 0)
            pltpu.sync_copy(
                rbuf, rst2.at[pl.ds((cid * N + b * BS) * HALF, BS * HALF)])
        return 0
    lax.fori_loop(0, NBLK, rblk, 0)
    plsc.subcore_barrier()

    def achunk(i, _):
        off = cid * (E // NC) + sid * EPA + i * K
        pltpu.sync_copy(edst.at[pl.ds(off, K)], dstb)
        pltpu.sync_copy(ef.at[pl.ds(off, K)], typb)
        wcp = pltpu.async_copy(wtabS.at[typb], wbuf, sem2)
        pltpu.async_copy(acc.at[dstb], sbuf, sem).wait()
        wcp.wait()

        def arow(q, _):
            for u in range(4):
                p = q * 4 + u
                abuf[pl.ds(p * AW, L)] = (
                    wbuf[p, pl.ds(HALF, L)] / sbuf[p, pl.ds(HALF, L)])
                abuf[pl.ds(p * AW + L, L)] = (
                    wbuf[p, pl.ds(HALF + L, L)] / sbuf[p, pl.ds(HALF + L, L)])
            return 0
        lax.fori_loop(0, K // 4, arow, 0)
        pltpu.sync_copy(abuf, attnf.at[pl.ds(off * AW, K * AW)])
        return 0
    lax.fori_loop(0, NCA, achunk, 0)


_sc_call = functools.partial(
    pl.kernel,
    out_type=[
        jax.ShapeDtypeStruct((NC * N * HALF,), jnp.float32),
        jax.ShapeDtypeStruct((E * AW,), jnp.float32),
    ],
    mesh=plsc.VectorSubcoreMesh(core_axis_name="c", subcore_axis_name="s"),
    scratch_types=[
        pltpu.VMEM_SHARED((N, ACCW), jnp.float32),
        pltpu.VMEM_SHARED((NT, ACCW), jnp.float32),
        pltpu.VMEM((H, NT), jnp.float32),
        pltpu.VMEM((H * L,), jnp.float32),
        pltpu.VMEM((NT * ACCW,), jnp.float32),
        pltpu.VMEM((K,), jnp.int32),
        pltpu.VMEM((K,), jnp.int32),
        pltpu.VMEM((K,), jnp.int32),
        pltpu.VMEM((K, HD), jnp.float32),
        pltpu.VMEM((K, ACCW), jnp.float32),
        pltpu.VMEM((K, ACCW), jnp.float32),
        pltpu.VMEM((K,), jnp.int32),
        pltpu.VMEM((K, ACCW), jnp.float32),
        pltpu.VMEM((K * AW,), jnp.float32),
        pltpu.VMEM((BS, ACCW), jnp.float32),
        pltpu.VMEM((BS * HALF,), jnp.float32),
        pltpu.SemaphoreType.DMA,
        pltpu.SemaphoreType.DMA,
        pltpu.SemaphoreType.DMA,
        pltpu.SemaphoreType.DMA,
    ],
)(_sc_body)


def kernel(feat, edge_index, e_feat, W, edge_emb_weight):
    fsall = _project(feat, W.T)
    embT = edge_emb_weight.T.astype(jnp.float32)
    rst2f, attnf = _sc_call(fsall, edge_index[0], edge_index[1], e_feat, embT)
    rr = rst2f.reshape(NC, N, HALF)
    rst = jnp.concatenate([rr[0], rr[1]], axis=1).reshape(N, H, D_OUT)
    attn32 = _transpose_attn(attnf.reshape(E, AW))
    attn = jnp.transpose(
        attn32.reshape(H, DROP_BLOCKS, 1, E), (3, 0, 1, 2))
    return (rst, attn)
